# baseline (speedup 1.0000x reference)
import functools

import jax
import jax.numpy as jnp
from jax.experimental import pallas as pl
from jax.experimental.pallas import tpu as pltpu

_B, _S, _D, _E = 4, 4096, 4096, 64
_BT = 512
_NBUF = 3


def _outer(x_hbm, wt_ref, b_ref, mask_hbm, prob_hbm):
    def _inner(x_blk, mask_blk, prob_blk):
        logits = jnp.dot(x_blk[...], wt_ref[...],
                         preferred_element_type=jnp.float32)
        logits = logits + b_ref[...]
        lt = logits.T
        m = jnp.max(lt, axis=0)
        row = jax.lax.broadcasted_iota(jnp.int32, lt.shape, 0)
        idx = jnp.min(jnp.where(lt == m[None, :], row, _E), axis=0)
        denom = jnp.sum(jnp.exp(lt - m[None, :]), axis=0)
        mask_blk[0, 0, :] = idx
        prob_blk[0, 0, :] = 1.0 / denom

    n_blk = (_B * _S) // _BT
    pltpu.emit_pipeline(
        _inner,
        grid=(n_blk,),
        in_specs=[
            pl.BlockSpec((_BT, _D), lambda i: (i, 0),
                         pipeline_mode=pl.Buffered(buffer_count=_NBUF)),
        ],
        out_specs=[
            pl.BlockSpec((1, 1, _BT), lambda i: (i, 0, 0)),
            pl.BlockSpec((1, 1, _BT), lambda i: (i, 0, 0)),
        ],
    )(x_hbm, mask_hbm, prob_hbm)


@functools.partial(jax.jit, static_argnums=())
def kernel(input_tokens, W, b):
    n_tok = _B * _S
    n_blk = n_tok // _BT
    x = input_tokens.reshape(n_tok, _D)
    wt = W.T
    b2 = b.reshape(1, _E)
    mask3, prob3 = pl.pallas_call(
        _outer,
        in_specs=[
            pl.BlockSpec(memory_space=pltpu.MemorySpace.HBM),
            pl.BlockSpec((_D, _E), lambda: (0, 0)),
            pl.BlockSpec((1, _E), lambda: (0, 0)),
        ],
        out_specs=[
            pl.BlockSpec(memory_space=pltpu.MemorySpace.HBM),
            pl.BlockSpec(memory_space=pltpu.MemorySpace.HBM),
        ],
        out_shape=[
            jax.ShapeDtypeStruct((n_blk, 1, _BT), jnp.int32),
            jax.ShapeDtypeStruct((n_blk, 1, _BT), jnp.float32),
        ],
    )(x, wt, b2)
    token_mask = mask3.reshape(_B, _S)
    expert_probs = prob3.reshape(_B, _S)
    capacity_loss = jnp.asarray(0.0, dtype=jnp.float32)
    return (token_mask, expert_probs, capacity_loss)
